# baseline (speedup 1.0000x reference)
import functools

import jax
import jax.numpy as jnp
from jax import lax
from jax.experimental import pallas as pl
from jax.experimental.pallas import tpu as pltpu
from jax.experimental.pallas import tpu_sc as plsc

_K = 15000
_N = 301056
_NW = 32
_CH = 9408
_NCHUNK = _N // _CH
_VPC = _CH // 16
_UNROLL = 12
_NBINS = 65536
_ROWS = 2352
_LANES = 128


def _order_i32(s):
    return s ^ ((s >> 31) & jnp.int32(0x7FFFFFFF))


def _suffix_select(load_group, ngroups, k_cur):
    lane = lax.iota(jnp.int32, 16)

    def body(g, carry):
        suffix_base, cnt_vec = carry
        gg = ngroups - 1 - g
        v = load_group(gg)
        csum = plsc.cumsum(v)
        gsum = jnp.sum(v)
        suffix_in = gsum - csum + v
        cond = (suffix_base + suffix_in) >= k_cur
        return suffix_base + gsum, cnt_vec + cond.astype(jnp.int32)

    _, cnt_vec = lax.fori_loop(
        0, ngroups, body, (jnp.int32(0), jnp.zeros((16,), jnp.int32))
    )
    b = jnp.sum(cnt_vec) - 1

    def body2(g, acc):
        v = load_group(g)
        bidx = lane + g * 16
        return acc + jnp.where(bidx > b, v, 0)

    above_vec = lax.fori_loop(0, ngroups, body2, jnp.zeros((16,), jnp.int32))
    return b, jnp.sum(above_vec)


def _sc_select_body(x_hbm, out_hbm, data_v, hist_v, ss_v, out_v):
    wid = lax.axis_index("s") * 2 + lax.axis_index("c")
    row0 = wid * _N
    zeros16 = jnp.zeros((16,), jnp.int32)
    ones16 = jnp.ones((16,), jnp.int32)
    lane = lax.iota(jnp.int32, 16)

    def zero_hist():
        def zbody(i, _):
            for u in range(8):
                hist_v[pl.ds((i * 8 + u) * 16, 16)] = zeros16
            return 0

        lax.fori_loop(0, _NBINS // (16 * 8), zbody, 0)

    def sweep(update_fn):
        def cbody(c, _):
            pltpu.sync_copy(x_hbm.at[pl.ds(row0 + c * _CH, _CH)], data_v)

            def jbody(j, _):
                for u in range(_UNROLL):
                    v = data_v[pl.ds((j * _UNROLL + u) * 16, 16)]
                    s = plsc.bitcast(v, jnp.int32)
                    update_fn(_order_i32(s))
                return 0

            lax.fori_loop(0, _VPC // _UNROLL, jbody, 0)
            return 0

        lax.fori_loop(0, _NCHUNK, cbody, 0)

    def scan_hist(k_cur):
        def sbody(S, _):
            acc = zeros16
            for j in range(16):
                acc = acc + hist_v[pl.ds(S * 256 + j * 16, 16)]
            ss_v[pl.ds(S * 16, 16)] = zeros16 + jnp.sum(acc)
            return 0

        lax.fori_loop(0, 256, sbody, 0)

        sup, above_sup = _suffix_select(
            lambda g: plsc.load_gather(ss_v, [(lane + g * 16) * 16]), 16, k_cur
        )
        sub, above_sub = _suffix_select(
            lambda g: hist_v[pl.ds(sup * 256 + g * 16, 16)],
            16,
            k_cur - above_sup,
        )
        return sup * 256 + sub, k_cur - above_sup - above_sub

    with jax.named_scope("zero1"):
        zero_hist()

    def upd1(ordx):
        b = (ordx >> 16) + jnp.int32(32768)
        plsc.addupdate_scatter(hist_v, [b], ones16)

    with jax.named_scope("sweep1"):
        sweep(upd1)
    with jax.named_scope("scan1"):
        b1, k2 = scan_hist(jnp.int32(_K))
    hi16 = b1 - 32768

    with jax.named_scope("zero2"):
        zero_hist()

    def upd2(ordx):
        sel = (ordx >> 16) == hi16
        b = ordx & jnp.int32(0xFFFF)
        plsc.addupdate_scatter(hist_v, [b], ones16, mask=sel)

    with jax.named_scope("sweep2"):
        sweep(upd2)
    with jax.named_scope("scan2"):
        b2, _ = scan_hist(k2)

    tbits = (hi16 << 16) | b2
    out_v[...] = zeros16 + tbits
    pltpu.sync_copy(out_v, out_hbm.at[wid])


@functools.partial(
    pl.kernel,
    out_type=jax.ShapeDtypeStruct((_NW, 16), jnp.int32),
    mesh=plsc.VectorSubcoreMesh(core_axis_name="c", subcore_axis_name="s"),
    compiler_params=pltpu.CompilerParams(needs_layout_passes=False),
    scratch_types=[
        pltpu.VMEM((_CH,), jnp.float32),
        pltpu.VMEM((_NBINS,), jnp.int32),
        pltpu.VMEM((256 * 16,), jnp.int32),
        pltpu.VMEM((16,), jnp.int32),
    ],
)
def _sc_select(x_hbm, out_hbm, data_v, hist_v, ss_v, out_v):
    _sc_select_body(x_hbm, out_hbm, data_v, hist_v, ss_v, out_v)


def _tc_mask_body(t_smem, x_ref, o_ref):
    x = x_ref[0]
    s = jax.lax.bitcast_convert_type(x, jnp.int32)
    ordx = _order_i32(s)
    t = t_smem[pl.program_id(0)]
    o_ref[0] = jnp.where(ordx >= t, x, jnp.float32(0.0))


@jax.jit
def kernel(x):
    B = x.shape[0]
    tbits = _sc_select(x.reshape(-1))
    out = pl.pallas_call(
        _tc_mask_body,
        grid_spec=pltpu.PrefetchScalarGridSpec(
            num_scalar_prefetch=1,
            grid=(B,),
            in_specs=[pl.BlockSpec((1, _ROWS, _LANES), lambda i, t: (i, 0, 0))],
            out_specs=pl.BlockSpec((1, _ROWS, _LANES), lambda i, t: (i, 0, 0)),
        ),
        out_shape=jax.ShapeDtypeStruct((B, _ROWS, _LANES), jnp.float32),
    )(tbits[:, 0], x.reshape(B, _ROWS, _LANES))
    return out.reshape(x.shape)


# speedup vs baseline: 1.0898x; 1.0898x over previous
import functools

import jax
import jax.numpy as jnp
from jax import lax
from jax.experimental import pallas as pl
from jax.experimental.pallas import tpu as pltpu
from jax.experimental.pallas import tpu_sc as plsc

_K = 15000
_N = 301056
_NW = 32
_CH = 9408
_NCHUNK = _N // _CH
_VPC = _CH // 16
_UNROLL = 12
_NBINS = 65536
_ROWS = 2352
_LANES = 128


def _order_i32(s):
    return s ^ ((s >> 31) & jnp.int32(0x7FFFFFFF))


def _suffix_select(load_group, ngroups, k_cur):
    lane = lax.iota(jnp.int32, 16)

    def body(g, carry):
        suffix_base, cnt_vec = carry
        gg = ngroups - 1 - g
        v = load_group(gg)
        csum = plsc.cumsum(v)
        gsum = jnp.sum(v)
        suffix_in = gsum - csum + v
        cond = (suffix_base + suffix_in) >= k_cur
        return suffix_base + gsum, cnt_vec + cond.astype(jnp.int32)

    _, cnt_vec = lax.fori_loop(
        0, ngroups, body, (jnp.int32(0), jnp.zeros((16,), jnp.int32))
    )
    b = jnp.sum(cnt_vec) - 1

    def body2(g, acc):
        v = load_group(g)
        bidx = lane + g * 16
        return acc + jnp.where(bidx > b, v, 0)

    above_vec = lax.fori_loop(0, ngroups, body2, jnp.zeros((16,), jnp.int32))
    return b, jnp.sum(above_vec)


def _sc_select_body(x_hbm, out_hbm, data_v, hist_v, ss_v, out_v):
    wid = lax.axis_index("s") * 2 + lax.axis_index("c")
    row0 = wid * _N
    zeros16 = jnp.zeros((16,), jnp.int32)
    ones16 = jnp.ones((16,), jnp.int32)
    lane = lax.iota(jnp.int32, 16)

    def zero_hist():
        def zbody(i, _):
            for u in range(8):
                hist_v[pl.ds((i * 8 + u) * 16, 16)] = zeros16
            return 0

        lax.fori_loop(0, _NBINS // (16 * 8), zbody, 0)

    def sweep(update_fn):
        def cbody(c, _):
            pltpu.sync_copy(x_hbm.at[pl.ds(row0 + c * _CH, _CH)], data_v)

            def jbody(j, _):
                for u in range(_UNROLL):
                    v = data_v[pl.ds((j * _UNROLL + u) * 16, 16)]
                    s = plsc.bitcast(v, jnp.int32)
                    update_fn(_order_i32(s))
                return 0

            lax.fori_loop(0, _VPC // _UNROLL, jbody, 0)
            return 0

        lax.fori_loop(0, _NCHUNK, cbody, 0)

    def scan_hist(k_cur):
        def sbody(S, _):
            acc = zeros16
            for j in range(16):
                acc = acc + hist_v[pl.ds(S * 256 + j * 16, 16)]
            ss_v[pl.ds(S * 16, 16)] = zeros16 + jnp.sum(acc)
            return 0

        lax.fori_loop(0, 256, sbody, 0)

        sup, above_sup = _suffix_select(
            lambda g: plsc.load_gather(ss_v, [(lane + g * 16) * 16]), 16, k_cur
        )
        sub, above_sub = _suffix_select(
            lambda g: hist_v[pl.ds(sup * 256 + g * 16, 16)],
            16,
            k_cur - above_sup,
        )
        return sup * 256 + sub, k_cur - above_sup - above_sub

    with jax.named_scope("zero1"):
        zero_hist()

    def upd1(ordx):
        b = (ordx >> 16) + jnp.int32(32768)
        plsc.addupdate_scatter(hist_v, [b], ones16)

    with jax.named_scope("sweep1"):
        sweep(upd1)
    with jax.named_scope("scan1"):
        b1, k2 = scan_hist(jnp.int32(_K))
    hi16 = b1 - 32768

    with jax.named_scope("zero2"):
        zero_hist()

    def upd2(ordx):
        sel = (ordx >> 16) == hi16
        b = ordx & jnp.int32(0xFFFF)
        plsc.addupdate_scatter(hist_v, [b], ones16, mask=sel)

    with jax.named_scope("sweep2"):
        sweep(upd2)
    with jax.named_scope("scan2"):
        b2, _ = scan_hist(k2)

    tbits = (hi16 << 16) | b2
    out_v[...] = zeros16 + tbits
    pltpu.sync_copy(out_v, out_hbm.at[wid])


@functools.partial(
    pl.kernel,
    out_type=jax.ShapeDtypeStruct((_NW, 16), jnp.int32),
    mesh=plsc.VectorSubcoreMesh(core_axis_name="c", subcore_axis_name="s"),
    compiler_params=pltpu.CompilerParams(needs_layout_passes=False),
    scratch_types=[
        pltpu.VMEM((_CH,), jnp.float32),
        pltpu.VMEM((_NBINS,), jnp.int32),
        pltpu.VMEM((256 * 16,), jnp.int32),
        pltpu.VMEM((16,), jnp.int32),
    ],
)
def _sc_select(x_hbm, out_hbm, data_v, hist_v, ss_v, out_v):
    _sc_select_body(x_hbm, out_hbm, data_v, hist_v, ss_v, out_v)


def _tc_mask_body(t_smem, x_ref, o_ref):
    x = x_ref[0]
    s = jax.lax.bitcast_convert_type(x, jnp.int32)
    ordx = _order_i32(s)
    t = t_smem[pl.program_id(0)]
    o_ref[0] = jnp.where(ordx >= t, x, jnp.float32(0.0))


@jax.jit
def kernel(x):
    B, C, H, W = x.shape
    tbits = _sc_select(x.reshape(-1))
    return pl.pallas_call(
        _tc_mask_body,
        grid_spec=pltpu.PrefetchScalarGridSpec(
            num_scalar_prefetch=1,
            grid=(B,),
            in_specs=[pl.BlockSpec((1, C, H, W), lambda i, t: (i, 0, 0, 0))],
            out_specs=pl.BlockSpec((1, C, H, W), lambda i, t: (i, 0, 0, 0)),
        ),
        out_shape=jax.ShapeDtypeStruct((B, C, H, W), jnp.float32),
    )(tbits[:, 0], x)


# speedup vs baseline: 1.8614x; 1.7080x over previous
import functools

import jax
import jax.numpy as jnp
from jax import lax
from jax.experimental import pallas as pl
from jax.experimental.pallas import tpu as pltpu
from jax.experimental.pallas import tpu_sc as plsc

_K = 15000
_N = 301056
_NW = 32
_CH = 9408
_NCHUNK = _N // _CH
_VPC = _CH // 16
_UNROLL = 12
_NBINS = 65536
_ROWS = 2352
_LANES = 128


def _order_i32(s):
    return s ^ ((s >> 31) & jnp.int32(0x7FFFFFFF))


def _suffix_select(load_group, ngroups, k_cur):
    lane = lax.iota(jnp.int32, 16)

    def body(g, carry):
        suffix_base, cnt_vec = carry
        gg = ngroups - 1 - g
        v = load_group(gg)
        csum = plsc.cumsum(v)
        gsum = jnp.sum(v)
        suffix_in = gsum - csum + v
        cond = (suffix_base + suffix_in) >= k_cur
        return suffix_base + gsum, cnt_vec + cond.astype(jnp.int32)

    _, cnt_vec = lax.fori_loop(
        0, ngroups, body, (jnp.int32(0), jnp.zeros((16,), jnp.int32))
    )
    b = jnp.sum(cnt_vec) - 1

    def body2(g, acc):
        v = load_group(g)
        bidx = lane + g * 16
        return acc + jnp.where(bidx > b, v, 0)

    above_vec = lax.fori_loop(0, ngroups, body2, jnp.zeros((16,), jnp.int32))
    return b, jnp.sum(above_vec)


def _sc_select_body(x_hbm, out_hbm, data_v, hist_v, ss_v, out_v):
    wid = lax.axis_index("s") * 2 + lax.axis_index("c")
    row0 = wid * _N
    zeros16 = jnp.zeros((16,), jnp.int32)
    ones16 = jnp.ones((16,), jnp.int32)
    lane = lax.iota(jnp.int32, 16)

    def zero_hist():
        @plsc.parallel_loop(0, _NBINS, 16, unroll=8)
        def _(i):
            hist_v[pl.ds(i, 16)] = zeros16

    def sweep(update_fn):
        def cbody(c, _):
            pltpu.sync_copy(x_hbm.at[pl.ds(row0 + c * _CH, _CH)], data_v)

            @plsc.parallel_loop(0, _CH, 16, unroll=_UNROLL)
            def _(i):
                v = data_v[pl.ds(i, 16)]
                s = plsc.bitcast(v, jnp.int32)
                update_fn(_order_i32(s))

            return 0

        lax.fori_loop(0, _NCHUNK, cbody, 0)

    def scan_hist(k_cur):
        def sbody(S, _):
            acc = zeros16
            for j in range(16):
                acc = acc + hist_v[pl.ds(S * 256 + j * 16, 16)]
            ss_v[pl.ds(S * 16, 16)] = zeros16 + jnp.sum(acc)
            return 0

        lax.fori_loop(0, 256, sbody, 0)

        sup, above_sup = _suffix_select(
            lambda g: plsc.load_gather(ss_v, [(lane + g * 16) * 16]), 16, k_cur
        )
        sub, above_sub = _suffix_select(
            lambda g: hist_v[pl.ds(sup * 256 + g * 16, 16)],
            16,
            k_cur - above_sup,
        )
        return sup * 256 + sub, k_cur - above_sup - above_sub

    with jax.named_scope("zero1"):
        zero_hist()

    def upd1(ordx):
        b = (ordx >> 16) + jnp.int32(32768)
        plsc.addupdate_scatter(hist_v, [b], ones16)

    with jax.named_scope("sweep1"):
        sweep(upd1)
    with jax.named_scope("scan1"):
        b1, k2 = scan_hist(jnp.int32(_K))
    hi16 = b1 - 32768

    with jax.named_scope("zero2"):
        zero_hist()

    def upd2(ordx):
        sel = (ordx >> 16) == hi16
        b = ordx & jnp.int32(0xFFFF)
        plsc.addupdate_scatter(hist_v, [b], ones16, mask=sel)

    with jax.named_scope("sweep2"):
        sweep(upd2)
    with jax.named_scope("scan2"):
        b2, _ = scan_hist(k2)

    tbits = (hi16 << 16) | b2
    out_v[...] = zeros16 + tbits
    pltpu.sync_copy(out_v, out_hbm.at[wid])


@functools.partial(
    pl.kernel,
    out_type=jax.ShapeDtypeStruct((_NW, 16), jnp.int32),
    mesh=plsc.VectorSubcoreMesh(core_axis_name="c", subcore_axis_name="s"),
    compiler_params=pltpu.CompilerParams(needs_layout_passes=False),
    scratch_types=[
        pltpu.VMEM((_CH,), jnp.float32),
        pltpu.VMEM((_NBINS,), jnp.int32),
        pltpu.VMEM((256 * 16,), jnp.int32),
        pltpu.VMEM((16,), jnp.int32),
    ],
)
def _sc_select(x_hbm, out_hbm, data_v, hist_v, ss_v, out_v):
    _sc_select_body(x_hbm, out_hbm, data_v, hist_v, ss_v, out_v)


def _tc_mask_body(t_smem, x_ref, o_ref):
    x = x_ref[0]
    s = jax.lax.bitcast_convert_type(x, jnp.int32)
    ordx = _order_i32(s)
    t = t_smem[pl.program_id(0)]
    o_ref[0] = jnp.where(ordx >= t, x, jnp.float32(0.0))


@jax.jit
def kernel(x):
    B, C, H, W = x.shape
    tbits = _sc_select(x.reshape(-1))
    return pl.pallas_call(
        _tc_mask_body,
        grid_spec=pltpu.PrefetchScalarGridSpec(
            num_scalar_prefetch=1,
            grid=(B,),
            in_specs=[pl.BlockSpec((1, C, H, W), lambda i, t: (i, 0, 0, 0))],
            out_specs=pl.BlockSpec((1, C, H, W), lambda i, t: (i, 0, 0, 0)),
        ),
        out_shape=jax.ShapeDtypeStruct((B, C, H, W), jnp.float32),
    )(tbits[:, 0], x)


# speedup vs baseline: 2.4497x; 1.3161x over previous
import functools

import jax
import jax.numpy as jnp
from jax import lax
from jax.experimental import pallas as pl
from jax.experimental.pallas import tpu as pltpu
from jax.experimental.pallas import tpu_sc as plsc

_K = 15000
_NW = 32
_NCH = 2
_NCHUNK = 96 // _NCH
_UNROLL = 8
_NBINS = 65536


def _order_i32(s):
    return s ^ ((s >> 31) & jnp.int32(0x7FFFFFFF))


def _suffix_select(load_group, ngroups, k_cur):
    lane = lax.iota(jnp.int32, 16)

    def body(g, carry):
        suffix_base, cnt_vec = carry
        gg = ngroups - 1 - g
        v = load_group(gg)
        csum = plsc.cumsum(v)
        gsum = jnp.sum(v)
        suffix_in = gsum - csum + v
        cond = (suffix_base + suffix_in) >= k_cur
        return suffix_base + gsum, cnt_vec + cond.astype(jnp.int32)

    _, cnt_vec = lax.fori_loop(
        0, ngroups, body, (jnp.int32(0), jnp.zeros((16,), jnp.int32))
    )
    b = jnp.sum(cnt_vec) - 1

    def body2(g, acc):
        v = load_group(g)
        bidx = lane + g * 16
        return acc + jnp.where(bidx > b, v, 0)

    above_vec = lax.fori_loop(0, ngroups, body2, jnp.zeros((16,), jnp.int32))
    return b, jnp.sum(above_vec)


def _sc_select_body(x_hbm, out_hbm, data_v, data2_v, hist_v, ss_v, out_v, sem_a, sem_b):
    wid = lax.axis_index("s") * 2 + lax.axis_index("c")
    zeros16 = jnp.zeros((16,), jnp.int32)
    ones16 = jnp.ones((16,), jnp.int32)
    lane = lax.iota(jnp.int32, 16)

    def zero_hist():
        @plsc.parallel_loop(0, _NBINS, 16, unroll=8)
        def _(i):
            hist_v[pl.ds(i, 16)] = zeros16

    hi8 = lane >= 8

    def sweep(update_fn):
        bufs = (data_v, data2_v)
        sems = (sem_a, sem_b)

        def start(c, b):
            pltpu.async_copy(x_hbm.at[wid, pl.ds(c * _NCH, _NCH)], bufs[b], sems[b])

        def wait(b):
            pltpu.make_async_copy(
                x_hbm.at[wid, pl.ds(0, _NCH)], bufs[b], sems[b]
            ).wait()

        start(0, 0)
        start(1, 1)

        def gbody(g, _):
            for b in range(2):
                c = g * 2 + b
                wait(b)
                buf = bufs[b]
                for ch in range(_NCH):

                    @plsc.parallel_loop(0, 56, 1, unroll=_UNROLL)
                    def _(r):
                        for cw, m in ((0, None), (16, None), (32, None), (40, hi8)):
                            v = buf[ch, r, pl.ds(cw, 16)]
                            s = plsc.bitcast(v, jnp.int32)
                            update_fn(_order_i32(s), m)

                @pl.when(c + 2 < _NCHUNK)
                def _():
                    start(c + 2, b)

            return 0

        lax.fori_loop(0, _NCHUNK // 2, gbody, 0)

    def scan_hist(k_cur):
        def sbody(S, _):
            acc = zeros16
            for j in range(16):
                acc = acc + hist_v[pl.ds(S * 256 + j * 16, 16)]
            ss_v[pl.ds(S * 16, 16)] = zeros16 + jnp.sum(acc)
            return 0

        lax.fori_loop(0, 256, sbody, 0)

        sup, above_sup = _suffix_select(
            lambda g: plsc.load_gather(ss_v, [(lane + g * 16) * 16]), 16, k_cur
        )
        sub, above_sub = _suffix_select(
            lambda g: hist_v[pl.ds(sup * 256 + g * 16, 16)],
            16,
            k_cur - above_sup,
        )
        return sup * 256 + sub, k_cur - above_sup - above_sub

    with jax.named_scope("zero1"):
        zero_hist()

    def upd1(ordx, m):
        b = (ordx >> 16) + jnp.int32(32768)
        plsc.addupdate_scatter(hist_v, [b], ones16, mask=m)

    with jax.named_scope("sweep1"):
        sweep(upd1)
    with jax.named_scope("scan1"):
        b1, k2 = scan_hist(jnp.int32(_K))
    hi16 = b1 - 32768

    with jax.named_scope("zero2"):
        zero_hist()

    def upd2(ordx, m):
        sel = (ordx >> 16) == hi16
        if m is not None:
            sel = jnp.logical_and(sel, m)
        b = ordx & jnp.int32(0xFFFF)
        plsc.addupdate_scatter(hist_v, [b], ones16, mask=sel)

    with jax.named_scope("sweep2"):
        sweep(upd2)
    with jax.named_scope("scan2"):
        b2, _ = scan_hist(k2)

    tbits = (hi16 << 16) | b2
    out_v[...] = zeros16 + tbits
    pltpu.sync_copy(out_v, out_hbm.at[wid])


@functools.partial(
    pl.kernel,
    out_type=jax.ShapeDtypeStruct((_NW, 16), jnp.int32),
    mesh=plsc.VectorSubcoreMesh(core_axis_name="c", subcore_axis_name="s"),
    compiler_params=pltpu.CompilerParams(needs_layout_passes=False),
    scratch_types=[
        pltpu.VMEM((_NCH, 56, 56), jnp.float32),
        pltpu.VMEM((_NCH, 56, 56), jnp.float32),
        pltpu.VMEM((_NBINS,), jnp.int32),
        pltpu.VMEM((256 * 16,), jnp.int32),
        pltpu.VMEM((16,), jnp.int32),
        pltpu.SemaphoreType.DMA,
        pltpu.SemaphoreType.DMA,
    ],
)
def _sc_select(x_hbm, out_hbm, data_v, data2_v, hist_v, ss_v, out_v, sem_a, sem_b):
    _sc_select_body(x_hbm, out_hbm, data_v, data2_v, hist_v, ss_v, out_v, sem_a, sem_b)


def _tc_mask_body(t_smem, x_ref, o_ref):
    x = x_ref[0]
    s = jax.lax.bitcast_convert_type(x, jnp.int32)
    ordx = _order_i32(s)
    t = t_smem[pl.program_id(0)]
    o_ref[0] = jnp.where(ordx >= t, x, jnp.float32(0.0))


@jax.jit
def kernel(x):
    B, C, H, W = x.shape
    tbits = _sc_select(x)
    return pl.pallas_call(
        _tc_mask_body,
        grid_spec=pltpu.PrefetchScalarGridSpec(
            num_scalar_prefetch=1,
            grid=(B,),
            in_specs=[pl.BlockSpec((1, C, H, W), lambda i, t: (i, 0, 0, 0))],
            out_specs=pl.BlockSpec((1, C, H, W), lambda i, t: (i, 0, 0, 0)),
        ),
        out_shape=jax.ShapeDtypeStruct((B, C, H, W), jnp.float32),
    )(tbits[:, 0], x)
